# baseline (device time: 11766 ns/iter reference)
import jax
import jax.numpy as jnp
from jax import lax
from jax.experimental import pallas as pl
from jax.experimental.pallas import tpu as pltpu

N_DEV = 4
G = 8


def kernel(x, dy, gamma):
    m, d = x.shape
    bm = m // G
    H = G // 2

    def body(x_ref, dy_ref, out_ref, comm_ref, send_sems, recv_sems):
        g = pl.program_id(0)
        my = lax.axis_index("i")
        barrier_sem = pltpu.get_barrier_semaphore()

        def make_rdma(base, k):
            sem = base // 4 * 3 + k - 1
            return pltpu.make_async_remote_copy(
                src_ref=comm_ref.at[base],
                dst_ref=comm_ref.at[base + k],
                send_sem=send_sems.at[sem],
                recv_sem=recv_sems.at[sem],
                device_id=((my + k) % N_DEV,),
                device_id_type=pl.DeviceIdType.MESH,
            )

        @pl.when(g == 0)
        def _():
            for k in range(1, N_DEV):
                pl.semaphore_signal(
                    barrier_sem, inc=1,
                    device_id=((my + k) % N_DEV,),
                    device_id_type=pl.DeviceIdType.MESH,
                )

        xv = x_ref[:, :].astype(jnp.bfloat16)
        dyv = dy_ref[:, :].astype(jnp.bfloat16)
        sx = jnp.sum(xv, axis=1, keepdims=True, dtype=jnp.float32)
        sxx = jnp.sum(xv * xv, axis=1, keepdims=True, dtype=jnp.float32)
        mu = sx * (1.0 / d)
        var = sxx * (1.0 / d) - mu * mu
        rstd = lax.rsqrt(var + 1e-5)
        t = dyv * ((xv - mu.astype(jnp.bfloat16)) * rstd.astype(jnp.bfloat16))
        dgamma = jnp.sum(t, axis=0, dtype=jnp.float32)
        dbeta = jnp.sum(dyv, axis=0, dtype=jnp.float32)
        part = jnp.stack([dgamma, dbeta])

        @pl.when(g == 0)
        def _():
            comm_ref[0, :, :] = part

        @pl.when((g > 0) & (g < H))
        def _():
            comm_ref[0, :, :] += part

        @pl.when(g == H)
        def _():
            comm_ref[4, :, :] = part

        @pl.when(g > H)
        def _():
            comm_ref[4, :, :] += part

        @pl.when(g == H - 1)
        def _():
            pl.semaphore_wait(barrier_sem, N_DEV - 1)
            for k in range(1, N_DEV):
                make_rdma(0, k).start()

        @pl.when(g == G - 1)
        def _():
            for k in range(1, N_DEV):
                make_rdma(4, k).start()
            for base in (0, 4):
                for k in range(1, N_DEV):
                    make_rdma(base, k).wait_recv()
            out_ref[:, :] = (
                (comm_ref[0] + comm_ref[1])
                + (comm_ref[2] + comm_ref[3])
                + (comm_ref[4] + comm_ref[5])
                + (comm_ref[6] + comm_ref[7])
            )
            for base in (0, 4):
                for k in range(1, N_DEV):
                    make_rdma(base, k).wait_send()

    return pl.pallas_call(
        body,
        grid=(G,),
        out_shape=jax.ShapeDtypeStruct((2, d), jnp.float32),
        in_specs=[
            pl.BlockSpec((bm, d), lambda g: (g, 0)),
            pl.BlockSpec((bm, d), lambda g: (g, 0)),
        ],
        out_specs=pl.BlockSpec((2, d), lambda g: (0, 0)),
        scratch_shapes=[
            pltpu.VMEM((2 * N_DEV, 2, d), jnp.float32),
            pltpu.SemaphoreType.DMA((6,)),
            pltpu.SemaphoreType.DMA((6,)),
        ],
        compiler_params=pltpu.CompilerParams(collective_id=0),
    )(x, dy)


# device time: 11430 ns/iter; 1.0294x vs baseline; 1.0294x over previous
import jax
import jax.numpy as jnp
from jax import lax
from jax.experimental import pallas as pl
from jax.experimental.pallas import tpu as pltpu

N_DEV = 4
G = 2


def kernel(x, dy, gamma):
    m, d = x.shape
    bm = m // G
    H = G // 2

    def body(x_ref, dy_ref, out_ref, comm_ref, send_sems, recv_sems):
        g = pl.program_id(0)
        my = lax.axis_index("i")
        barrier_sem = pltpu.get_barrier_semaphore()

        def make_rdma(base, k):
            sem = base // 4 * 3 + k - 1
            return pltpu.make_async_remote_copy(
                src_ref=comm_ref.at[base],
                dst_ref=comm_ref.at[base + k],
                send_sem=send_sems.at[sem],
                recv_sem=recv_sems.at[sem],
                device_id=((my + k) % N_DEV,),
                device_id_type=pl.DeviceIdType.MESH,
            )

        @pl.when(g == 0)
        def _():
            for k in range(1, N_DEV):
                pl.semaphore_signal(
                    barrier_sem, inc=1,
                    device_id=((my + k) % N_DEV,),
                    device_id_type=pl.DeviceIdType.MESH,
                )

        xv = x_ref[:, :].astype(jnp.bfloat16)
        dyv = dy_ref[:, :].astype(jnp.bfloat16)
        sx = jnp.sum(xv, axis=1, keepdims=True, dtype=jnp.float32)
        sxx = jnp.sum(xv * xv, axis=1, keepdims=True, dtype=jnp.float32)
        mu = sx * (1.0 / d)
        var = sxx * (1.0 / d) - mu * mu
        rstd = lax.rsqrt(var + 1e-5)
        t = dyv * ((xv - mu.astype(jnp.bfloat16)) * rstd.astype(jnp.bfloat16))
        dgamma = jnp.sum(t, axis=0, dtype=jnp.float32)
        dbeta = jnp.sum(dyv, axis=0, dtype=jnp.float32)
        part = jnp.stack([dgamma, dbeta])

        @pl.when(g == 0)
        def _():
            comm_ref[0, :, :] = part

        @pl.when((g > 0) & (g < H))
        def _():
            comm_ref[0, :, :] += part

        @pl.when(g == H)
        def _():
            comm_ref[4, :, :] = part

        @pl.when(g > H)
        def _():
            comm_ref[4, :, :] += part

        @pl.when(g == H - 1)
        def _():
            pl.semaphore_wait(barrier_sem, N_DEV - 1)
            for k in range(1, N_DEV):
                make_rdma(0, k).start()

        @pl.when(g == G - 1)
        def _():
            for k in range(1, N_DEV):
                make_rdma(4, k).start()
            for base in (0, 4):
                for k in range(1, N_DEV):
                    make_rdma(base, k).wait_recv()
            out_ref[:, :] = (
                (comm_ref[0] + comm_ref[1])
                + (comm_ref[2] + comm_ref[3])
                + (comm_ref[4] + comm_ref[5])
                + (comm_ref[6] + comm_ref[7])
            )
            for base in (0, 4):
                for k in range(1, N_DEV):
                    make_rdma(base, k).wait_send()

    return pl.pallas_call(
        body,
        grid=(G,),
        out_shape=jax.ShapeDtypeStruct((2, d), jnp.float32),
        in_specs=[
            pl.BlockSpec((bm, d), lambda g: (g, 0)),
            pl.BlockSpec((bm, d), lambda g: (g, 0)),
        ],
        out_specs=pl.BlockSpec((2, d), lambda g: (0, 0)),
        scratch_shapes=[
            pltpu.VMEM((2 * N_DEV, 2, d), jnp.float32),
            pltpu.SemaphoreType.DMA((6,)),
            pltpu.SemaphoreType.DMA((6,)),
        ],
        compiler_params=pltpu.CompilerParams(collective_id=0),
    )(x, dy)
